# baseline (speedup 1.0000x reference)
import functools

import jax
import jax.numpy as jnp
from jax import lax
from jax.experimental import pallas as pl
from jax.experimental.pallas import tpu as pltpu
from jax.experimental.pallas import tpu_sc as plsc

VOCAB_ROWS = 1000000
EMBED_DIM = 64
BATCH = 4096
FIELDS = 26
B = BATCH * FIELDS
NC, NS = 2, 16
NW = NC * NS
B_PER_W = B // NW
N_CHUNKS = 16
CH = B_PER_W // N_CHUNKS
VREGS_PER_CHUNK = CH // 16

T_BLK = 32768
T_GRID = (VOCAB_ROWS + T_BLK - 1) // T_BLK


def _tc_transpose(table_t):
    def body(in_ref, out_ref):
        xt = in_ref[...].T
        out_ref[:, 0:EMBED_DIM] = xt[0:T_BLK // 2]
        out_ref[:, EMBED_DIM:128] = xt[T_BLK // 2:T_BLK]

    return pl.pallas_call(
        body,
        grid=(T_GRID,),
        in_specs=[pl.BlockSpec((EMBED_DIM, T_BLK), lambda g: (0, g))],
        out_specs=pl.BlockSpec((T_BLK // 2, 128), lambda g: (g, 0)),
        out_shape=jax.ShapeDtypeStruct((T_GRID * T_BLK // 2, 128), jnp.float32),
    )(table_t)


def _sc_gather(idx_flat, lin):
    mesh = plsc.VectorSubcoreMesh(core_axis_name="c", subcore_axis_name="s")

    @functools.partial(
        pl.kernel,
        mesh=mesh,
        out_type=jax.ShapeDtypeStruct((B, EMBED_DIM), jnp.float32),
        scratch_types=[
            pltpu.VMEM((B_PER_W,), jnp.int32),
            pltpu.VMEM((B_PER_W,), jnp.int32),
            pltpu.VMEM((CH, 128), jnp.float32),
            pltpu.VMEM((CH, 128), jnp.float32),
            pltpu.VMEM((CH, 128), jnp.float32),
            pltpu.VMEM((CH, EMBED_DIM), jnp.float32),
            pltpu.VMEM((CH, EMBED_DIM), jnp.float32),
            pltpu.SemaphoreType.DMA,
            pltpu.SemaphoreType.DMA,
            pltpu.SemaphoreType.DMA,
        ],
        compiler_params=pltpu.CompilerParams(
            needs_layout_passes=False, use_tc_tiling_on_sc=False),
    )
    def k(idx_hbm, lin_hbm, out_hbm, pair_v, half_v, buf0, buf1, buf2,
          ebuf0, ebuf1, sem0, sem1, sem2):
        wid = lax.axis_index("s") * NC + lax.axis_index("c")
        base = wid * B_PER_W
        pltpu.sync_copy(idx_hbm.at[pl.ds(base, B_PER_W)], pair_v)
        hb = T_BLK.bit_length() - 2
        for v in range(B_PER_W // 16):
            sl = pl.ds(v * 16, 16)
            raw = pair_v[sl]
            half_v[sl] = lax.bitwise_and(lax.shift_right_logical(raw, hb), 1)
            pair_v[sl] = lax.bitwise_or(
                lax.shift_left(lax.shift_right_logical(raw, hb + 1), hb),
                lax.bitwise_and(raw, T_BLK // 2 - 1))

        bufs = (buf0, buf1, buf2)
        ebufs = (ebuf0, ebuf1)
        sems = (sem0, sem1, sem2)
        lane0 = lax.iota(jnp.int32, 16)

        def extract(i, buf, ebuf):
            def grp_body(jj, _):
                jvec = jj * 16 + lane0
                hvec = half_v[pl.ds(i * CH + jj * 16, 16)]
                src0 = hvec * 64

                for mb in range(0, EMBED_DIM, 16):
                    for t in range(16):
                        colvec = mb + lax.bitwise_and(t + lane0, 15)
                        vals = plsc.load_gather(buf, [jvec, src0 + colvec])
                        plsc.store_scatter(ebuf, [jvec, colvec], vals)
                return 0
            lax.fori_loop(0, CH // 16, grp_body, 0)

        NB = 3
        cps = [None] * N_CHUNKS
        for p in range(NB - 1):
            cps[p] = pltpu.async_copy(
                lin_hbm.at[pair_v.at[pl.ds(p * CH, CH)]], bufs[p], sems[p])
        for i in range(N_CHUNKS):
            if i + NB - 1 < N_CHUNKS:
                j = i + NB - 1
                cps[j] = pltpu.async_copy(
                    lin_hbm.at[pair_v.at[pl.ds(j * CH, CH)]],
                    bufs[j % NB], sems[j % NB])
            cps[i].wait()
            extract(i, bufs[i % NB], ebufs[i % 2])
            pltpu.sync_copy(ebufs[i % 2], out_hbm.at[pl.ds(base + i * CH, CH)])

    return k(idx_flat, lin)


def kernel(inputs, embedding):
    table_t = embedding.T
    lin = _tc_transpose(table_t)
    idx = inputs.T.reshape(-1).astype(jnp.int32)
    out = _sc_gather(idx, lin)
    return out.reshape(FIELDS, BATCH, EMBED_DIM).transpose(1, 0, 2)


# speedup vs baseline: 1.2926x; 1.2926x over previous
import functools

import jax
import jax.numpy as jnp
from jax import lax
from jax.experimental import pallas as pl
from jax.experimental.pallas import tpu as pltpu
from jax.experimental.pallas import tpu_sc as plsc

VOCAB_ROWS = 1000000
EMBED_DIM = 64
BATCH = 4096
FIELDS = 26
B = BATCH * FIELDS
NC, NS = 2, 16
NW = NC * NS
B_PER_W = B // NW
N_CHUNKS = 16
CH = B_PER_W // N_CHUNKS
VREGS_PER_CHUNK = CH // 16

T_BLK = 32768
T_GRID = (VOCAB_ROWS + T_BLK - 1) // T_BLK


def _tc_transpose(table_t):
    def body(in_ref, out_ref):
        x = in_ref[...]
        pre = jnp.concatenate(
            [x[:, :T_BLK // 2], x[:, T_BLK // 2:]], axis=0)
        out_ref[...] = pre.T

    return pl.pallas_call(
        body,
        grid=(T_GRID,),
        in_specs=[pl.BlockSpec((EMBED_DIM, T_BLK), lambda g: (0, g))],
        out_specs=pl.BlockSpec((T_BLK // 2, 128), lambda g: (g, 0)),
        out_shape=jax.ShapeDtypeStruct((T_GRID * T_BLK // 2, 128), jnp.float32),
    )(table_t)


def _sc_gather(idx_flat, lin):
    mesh = plsc.VectorSubcoreMesh(core_axis_name="c", subcore_axis_name="s")

    @functools.partial(
        pl.kernel,
        mesh=mesh,
        out_type=jax.ShapeDtypeStruct((B, 128), jnp.float32),
        scratch_types=[
            pltpu.VMEM((B_PER_W,), jnp.int32),
            pltpu.VMEM((B_PER_W,), jnp.int32),
            pltpu.VMEM((CH, 128), jnp.float32),
            pltpu.VMEM((CH, 128), jnp.float32),
            pltpu.VMEM((CH, 128), jnp.float32),
            pltpu.VMEM((CH, 128), jnp.float32),
            pltpu.SemaphoreType.DMA,
            pltpu.SemaphoreType.DMA,
            pltpu.SemaphoreType.DMA,
        ],
        compiler_params=pltpu.CompilerParams(needs_layout_passes=False),
    )
    def k(idx_hbm, lin_hbm, out_hbm, pair_v, half_v, buf0, buf1, buf2,
          ebuf0, sem0, sem1, sem2):
        wid = lax.axis_index("s") * NC + lax.axis_index("c")
        base = wid * B_PER_W
        pltpu.sync_copy(idx_hbm.at[pl.ds(base, B_PER_W)], pair_v)
        hb = T_BLK.bit_length() - 2
        for v in range(B_PER_W // 16):
            sl = pl.ds(v * 16, 16)
            raw = pair_v[sl]
            half_v[sl] = lax.bitwise_and(lax.shift_right_logical(raw, hb), 1)
            pair_v[sl] = lax.bitwise_or(
                lax.shift_left(lax.shift_right_logical(raw, hb + 1), hb),
                lax.bitwise_and(raw, T_BLK // 2 - 1))

        bufs = (buf0, buf1, buf2)
        sems = (sem0, sem1, sem2)
        lane0 = lax.iota(jnp.int32, 16)

        def extract(i, buf, ebuf):
            def grp_body(jj, _):
                jvec = jj * 16 + lane0
                hvec = half_v[pl.ds(i * CH + jj * 16, 16)]
                src0 = hvec * 64

                for mb in range(0, EMBED_DIM, 16):
                    for t in range(16):
                        colvec = mb + lax.bitwise_and(t + lane0, 15)
                        vals = plsc.load_gather(buf, [jvec, src0 + colvec])
                        plsc.store_scatter(ebuf, [jvec, colvec], vals)
                return 0
            lax.fori_loop(0, CH // 16, grp_body, 0)

        NB = 3
        cps = [None] * N_CHUNKS
        for p in range(NB - 1):
            cps[p] = pltpu.async_copy(
                lin_hbm.at[pair_v.at[pl.ds(p * CH, CH)]], bufs[p], sems[p])
        for i in range(N_CHUNKS):
            if i + NB - 1 < N_CHUNKS:
                j = i + NB - 1
                cps[j] = pltpu.async_copy(
                    lin_hbm.at[pair_v.at[pl.ds(j * CH, CH)]],
                    bufs[j % NB], sems[j % NB])
            cps[i].wait()
            extract(i, bufs[i % NB], ebuf0)
            pltpu.sync_copy(ebuf0, out_hbm.at[pl.ds(base + i * CH, CH)])

    return k(idx_flat, lin)


def kernel(inputs, embedding):
    table_t = embedding.T
    lin = _tc_transpose(table_t)
    idx = inputs.T.reshape(-1).astype(jnp.int32)
    out = _sc_gather(idx, lin)
    return (out[:, :EMBED_DIM]
            .reshape(FIELDS, BATCH, EMBED_DIM)
            .transpose(1, 0, 2))


# speedup vs baseline: 1.3612x; 1.0531x over previous
import functools

import jax
import jax.numpy as jnp
from jax import lax
from jax.experimental import pallas as pl
from jax.experimental.pallas import tpu as pltpu
from jax.experimental.pallas import tpu_sc as plsc

VOCAB_ROWS = 1000000
EMBED_DIM = 64
BATCH = 4096
FIELDS = 26
B = BATCH * FIELDS
NC, NS = 2, 16
NW = NC * NS
B_PER_W = B // NW
N_CHUNKS = 26
CH = B_PER_W // N_CHUNKS
VREGS_PER_CHUNK = CH // 16

T_BLK = 32768
T_GRID = (VOCAB_ROWS + T_BLK - 1) // T_BLK


def _tc_transpose(table_t):
    def body(in_ref, out_ref):
        x = in_ref[...]
        pre = jnp.concatenate(
            [x[:, :T_BLK // 2], x[:, T_BLK // 2:]], axis=0)
        out_ref[...] = pre.T

    return pl.pallas_call(
        body,
        grid=(T_GRID,),
        in_specs=[pl.BlockSpec((EMBED_DIM, T_BLK), lambda g: (0, g))],
        out_specs=pl.BlockSpec((T_BLK // 2, 128), lambda g: (g, 0)),
        out_shape=jax.ShapeDtypeStruct((T_GRID * T_BLK // 2, 128), jnp.float32),
    )(table_t)


def _sc_gather(idx_flat, lin):
    mesh = plsc.VectorSubcoreMesh(core_axis_name="c", subcore_axis_name="s")

    @functools.partial(
        pl.kernel,
        mesh=mesh,
        out_type=jax.ShapeDtypeStruct((B, 128), jnp.float32),
        scratch_types=[
            pltpu.VMEM((B_PER_W,), jnp.int32),
            pltpu.VMEM((B_PER_W,), jnp.int32),
            pltpu.VMEM((CH, 128), jnp.float32),
            pltpu.VMEM((CH, 128), jnp.float32),
            pltpu.VMEM((CH, 128), jnp.float32),
            pltpu.VMEM((CH, 128), jnp.float32),
            pltpu.VMEM((CH, 128), jnp.float32),
            pltpu.SemaphoreType.DMA,
            pltpu.SemaphoreType.DMA,
            pltpu.SemaphoreType.DMA,
            pltpu.SemaphoreType.DMA,
            pltpu.SemaphoreType.DMA,
        ],
        compiler_params=pltpu.CompilerParams(needs_layout_passes=False),
    )
    def k(idx_hbm, lin_hbm, out_hbm, pair_v, half_v, buf0, buf1, buf2,
          ebuf0, ebuf1, sem0, sem1, sem2, osem0, osem1):
        wid = lax.axis_index("s") * NC + lax.axis_index("c")
        base = wid * B_PER_W
        pltpu.sync_copy(idx_hbm.at[pl.ds(base, B_PER_W)], pair_v)
        hb = T_BLK.bit_length() - 2
        for v in range(B_PER_W // 16):
            sl = pl.ds(v * 16, 16)
            raw = pair_v[sl]
            half_v[sl] = lax.bitwise_and(lax.shift_right_logical(raw, hb), 1)
            pair_v[sl] = lax.bitwise_or(
                lax.shift_left(lax.shift_right_logical(raw, hb + 1), hb),
                lax.bitwise_and(raw, T_BLK // 2 - 1))

        bufs = (buf0, buf1, buf2)
        ebufs = (ebuf0, ebuf1)
        sems = (sem0, sem1, sem2)
        osems = (osem0, osem1)
        lane0 = lax.iota(jnp.int32, 16)

        def extract(i, buf, ebuf):
            def grp_body(jj, _):
                jvec = jj * 16 + lane0
                hvec = half_v[pl.ds(i * CH + jj * 16, 16)]
                src0 = hvec * 64

                for mb in range(0, EMBED_DIM, 16):
                    for t in range(16):
                        colvec = mb + lax.bitwise_and(t + lane0, 15)
                        vals = plsc.load_gather(buf, [jvec, src0 + colvec])
                        plsc.store_scatter(ebuf, [jvec, colvec], vals)
                return 0
            lax.fori_loop(0, CH // 16, grp_body, 0)

        NB = 3
        cps = [None] * N_CHUNKS
        ocps = [None] * N_CHUNKS
        for p in range(NB - 1):
            cps[p] = pltpu.async_copy(
                lin_hbm.at[pair_v.at[pl.ds(p * CH, CH)]], bufs[p], sems[p])
        for i in range(N_CHUNKS):
            if i + NB - 1 < N_CHUNKS:
                j = i + NB - 1
                cps[j] = pltpu.async_copy(
                    lin_hbm.at[pair_v.at[pl.ds(j * CH, CH)]],
                    bufs[j % NB], sems[j % NB])
            cps[i].wait()
            if i >= 2:
                ocps[i - 2].wait()
            extract(i, bufs[i % NB], ebufs[i % 2])
            ocps[i] = pltpu.async_copy(
                ebufs[i % 2], out_hbm.at[pl.ds(base + i * CH, CH)],
                osems[i % 2])
        ocps[N_CHUNKS - 2].wait()
        ocps[N_CHUNKS - 1].wait()

    return k(idx_flat, lin)


def kernel(inputs, embedding):
    table_t = embedding.T
    lin = _tc_transpose(table_t)
    idx = inputs.T.reshape(-1).astype(jnp.int32)
    out = _sc_gather(idx, lin)
    return (out[:, :EMBED_DIM]
            .reshape(FIELDS, BATCH, EMBED_DIM)
            .transpose(1, 0, 2))
